# baseline (device time: 77686 ns/iter reference)
import jax
import jax.numpy as jnp
from jax import lax
from jax.experimental import pallas as pl
from jax.experimental.pallas import tpu as pltpu

N_DEV = 32
B = 2
SQ = 256
SKV = 256
H_PER = 4
DH = 64
D_MODEL = 512
ROWS_PER = SQ // N_DEV


def kernel(x, Wq, K_ext, V_ext, Wo):
    pos = lax.axis_index("i")
    K_sl = lax.dynamic_slice_in_dim(K_ext, pos * H_PER, H_PER, axis=2)
    V_sl = lax.dynamic_slice_in_dim(V_ext, pos * H_PER, H_PER, axis=2)

    def body(x_ref, wq_ref, k_ref, v_ref, wo_ref, out_ref,
             partial_ref, comm_ref, red_ref,
             send1, recv1, send2, recv2):
        my = lax.axis_index("i")

        barrier = pltpu.get_barrier_semaphore()
        for p in range(N_DEV):
            @pl.when(p != my)
            def _(p=p):
                pl.semaphore_signal(
                    barrier, inc=1,
                    device_id=(p,), device_id_type=pl.DeviceIdType.MESH,
                )
        pl.semaphore_wait(barrier, N_DEV - 1)

        qi = lax.broadcasted_iota(jnp.int32, (SQ, SKV), 0)
        ki = lax.broadcasted_iota(jnp.int32, (SQ, SKV), 1)
        mask = jnp.abs(qi - ki) <= 128

        wq = wq_ref[:, :]
        wo = wo_ref[:, :]
        for b in range(B):
            xb = x_ref[b]
            qb = jnp.dot(xb, wq, preferred_element_type=jnp.float32)
            ctxs = []
            for h in range(H_PER):
                qh = qb[:, h * DH:(h + 1) * DH]
                kh = k_ref[b, :, h, :]
                vh = v_ref[b, :, h, :]
                s = lax.dot_general(
                    qh, kh, (((1,), (1,)), ((), ())),
                    preferred_element_type=jnp.float32,
                ) * 0.125
                s = jnp.where(mask, s, -1e9)
                s = s - jnp.max(s, axis=1, keepdims=True)
                w = jnp.exp(s)
                w = w / jnp.sum(w, axis=1, keepdims=True)
                ctxs.append(jnp.dot(w, vh, preferred_element_type=jnp.float32))
            ctx_b = jnp.concatenate(ctxs, axis=1)
            pb = jnp.dot(ctx_b, wo, preferred_element_type=jnp.float32)
            for j in range(N_DEV):
                partial_ref[j, b, :, :] = pb[j * ROWS_PER:(j + 1) * ROWS_PER, :]

        sends1 = []
        for p in range(N_DEV):
            rdma = pltpu.make_async_remote_copy(
                src_ref=partial_ref.at[p],
                dst_ref=comm_ref.at[my],
                send_sem=send1.at[p],
                recv_sem=recv1.at[my],
                device_id=(p,), device_id_type=pl.DeviceIdType.MESH,
            )
            sends1.append(rdma)
            @pl.when(p != my)
            def _(rdma=rdma):
                rdma.start()

        comm_ref[pl.ds(my, 1)] = partial_ref[pl.ds(my, 1)]

        for i in range(N_DEV):
            rdma = pltpu.make_async_remote_copy(
                src_ref=partial_ref.at[i],
                dst_ref=comm_ref.at[i],
                send_sem=send1.at[i],
                recv_sem=recv1.at[i],
                device_id=(i,), device_id_type=pl.DeviceIdType.MESH,
            )
            @pl.when(i != my)
            def _(rdma=rdma):
                rdma.wait_recv()

        red_ref[:, :, :] = jnp.sum(comm_ref[:, :, :, :], axis=0)

        sends2 = []
        for p in range(N_DEV):
            rdma = pltpu.make_async_remote_copy(
                src_ref=red_ref,
                dst_ref=out_ref.at[:, pl.ds(my * ROWS_PER, ROWS_PER), :],
                send_sem=send2.at[p],
                recv_sem=recv2.at[my],
                device_id=(p,), device_id_type=pl.DeviceIdType.MESH,
            )
            sends2.append(rdma)
            @pl.when(p != my)
            def _(rdma=rdma):
                rdma.start()

        out_ref[:, pl.ds(my * ROWS_PER, ROWS_PER), :] = red_ref[:, :, :]

        for i in range(N_DEV):
            rdma = pltpu.make_async_remote_copy(
                src_ref=red_ref,
                dst_ref=out_ref.at[:, pl.ds(i * ROWS_PER, ROWS_PER), :],
                send_sem=send2.at[i],
                recv_sem=recv2.at[i],
                device_id=(i,), device_id_type=pl.DeviceIdType.MESH,
            )
            @pl.when(i != my)
            def _(rdma=rdma):
                rdma.wait_recv()

        for p in range(N_DEV):
            @pl.when(p != my)
            def _(p=p):
                sends1[p].wait_send()
                sends2[p].wait_send()

    return pl.pallas_call(
        body,
        out_shape=jax.ShapeDtypeStruct((B, SQ, D_MODEL), jnp.float32),
        in_specs=[pl.BlockSpec(memory_space=pltpu.VMEM)] * 5,
        out_specs=pl.BlockSpec(memory_space=pltpu.VMEM),
        scratch_shapes=[
            pltpu.VMEM((N_DEV, B, ROWS_PER, D_MODEL), jnp.float32),
            pltpu.VMEM((N_DEV, B, ROWS_PER, D_MODEL), jnp.float32),
            pltpu.VMEM((B, ROWS_PER, D_MODEL), jnp.float32),
            pltpu.SemaphoreType.DMA((N_DEV,)),
            pltpu.SemaphoreType.DMA((N_DEV,)),
            pltpu.SemaphoreType.DMA((N_DEV,)),
            pltpu.SemaphoreType.DMA((N_DEV,)),
        ],
        compiler_params=pltpu.CompilerParams(collective_id=0),
    )(x, Wq, K_sl, V_sl, Wo)
